# baseline (speedup 1.0000x reference)
import functools

import jax
import jax.numpy as jnp
from jax import lax
from jax.experimental import pallas as pl
from jax.experimental.pallas import tpu as pltpu
from jax.experimental.pallas import tpu_sc as plsc

DIM = 128
GAMMA = 1.0
L = 16


def _sc_info():
    try:
        info = plsc.get_sparse_core_info()
        return info.num_cores, info.num_subcores
    except Exception:
        return 2, 16


@functools.lru_cache(maxsize=None)
def _build_sc(R):
    NC, NS = _sc_info()
    NW = NC * NS
    assert R % NW == 0
    n_per_w = R // NW
    CH = min(256, n_per_w)
    n_chunks = n_per_w // CH
    NJ = DIM // L
    n_groups = CH // L

    mesh = plsc.VectorSubcoreMesh(core_axis_name="c", subcore_axis_name="s")

    @functools.partial(
        pl.kernel,
        out_type=jax.ShapeDtypeStruct((NW, L), jnp.float32),
        mesh=mesh,
        compiler_params=pltpu.CompilerParams(needs_layout_passes=False),
        scratch_types=[
            pltpu.VMEM((n_per_w,), jnp.int32),
            pltpu.VMEM((n_per_w,), jnp.int32),
            pltpu.VMEM((n_per_w,), jnp.float32),
            pltpu.VMEM((8,), jnp.int32),
            pltpu.VMEM((8, DIM), jnp.float32),
            pltpu.VMEM((CH, DIM), jnp.float32),
            pltpu.VMEM((CH, DIM), jnp.float32),
            pltpu.VMEM((L, L), jnp.float32),
            pltpu.SemaphoreType.DMA,
            pltpu.SemaphoreType.DMA,
        ],
    )
    def sc_kernel(table_hbm, pconf_hbm, aidx_hbm, pos_hbm, neg_hbm, out_hbm,
                  posidx_v, negidx_v, pconf_v, aidx_v, a_v,
                  posrows_v, negrows_v, comb_v, sem_p, sem_n):
        wid = lax.axis_index("s") * NC + lax.axis_index("c")
        base = wid * n_per_w

        pltpu.sync_copy(pos_hbm.at[pl.ds(base, n_per_w)], posidx_v)
        pltpu.sync_copy(neg_hbm.at[pl.ds(base, n_per_w)], negidx_v)
        pltpu.sync_copy(pconf_hbm.at[pl.ds(base, n_per_w)], pconf_v)
        pltpu.sync_copy(aidx_hbm, aidx_v)
        pltpu.async_copy(table_hbm.at[aidx_v], a_v, sem_p).wait()
        a_sl = [a_v[0, pl.ds(L * j, L)] for j in range(NJ)]

        lane = jnp.arange(L, dtype=jnp.int32)
        loss_vec = jnp.zeros((L,), jnp.float32)
        for c in range(n_chunks):
            cp = pltpu.async_copy(
                table_hbm.at[posidx_v.at[pl.ds(c * CH, CH)]], posrows_v, sem_p)
            cn = pltpu.async_copy(
                table_hbm.at[negidx_v.at[pl.ds(c * CH, CH)]], negrows_v, sem_n)
            cp.wait()
            cn.wait()

            def group_body(g, lvec):
                base_r = g * L
                for r in range(L):
                    rr = base_r + r
                    pacc = jnp.abs(a_sl[0] - posrows_v[rr, pl.ds(0, L)])
                    nacc = jnp.abs(a_sl[0] - negrows_v[rr, pl.ds(0, L)])
                    for j in range(1, NJ):
                        pacc = pacc + jnp.abs(
                            a_sl[j] - posrows_v[rr, pl.ds(L * j, L)])
                        nacc = nacc + jnp.abs(
                            a_sl[j] - negrows_v[rr, pl.ds(L * j, L)])
                    pcs = plsc.load_gather(
                        pconf_v, [jnp.full((L,), c * CH + rr, jnp.int32)])
                    comb_v[r] = pcs * pacc - nacc
                colsum = plsc.load_gather(
                    comb_v, [lane, jnp.zeros((L,), jnp.int32)])
                for j in range(1, L):
                    colsum = colsum + plsc.load_gather(
                        comb_v, [lane, jnp.full((L,), j, jnp.int32)])
                return lvec + jnp.maximum(GAMMA + colsum, jnp.float32(0.0))

            loss_vec = lax.fori_loop(0, n_groups, group_body, loss_vec)

        comb_v[0] = loss_vec
        pltpu.sync_copy(comb_v.at[0], out_hbm.at[wid])

    return sc_kernel


def kernel(rel_emb, pconfi, rel_a, rel_pos, rel_neg):
    R = rel_pos.shape[0]
    sc = _build_sc(R)
    aidx = jnp.full((8,), rel_a, jnp.int32)
    partials = sc(rel_emb, pconfi, aidx,
                  rel_pos.astype(jnp.int32), rel_neg.astype(jnp.int32))
    return jnp.sum(partials)


# speedup vs baseline: 1.0733x; 1.0733x over previous
import functools

import jax
import jax.numpy as jnp
from jax import lax
from jax.experimental import pallas as pl
from jax.experimental.pallas import tpu as pltpu
from jax.experimental.pallas import tpu_sc as plsc

DIM = 128
GAMMA = 1.0
L = 16


def _sc_info():
    try:
        info = plsc.get_sparse_core_info()
        return info.num_cores, info.num_subcores
    except Exception:
        return 2, 16


@functools.lru_cache(maxsize=None)
def _build_sc(R):
    NC, NS = _sc_info()
    NW = NC * NS
    assert R % NW == 0
    n_per_w = R // NW
    CH = min(128, n_per_w)
    n_chunks = n_per_w // CH
    NBUF = min(2, n_chunks)
    NJ = DIM // L
    n_groups = CH // L

    mesh = plsc.VectorSubcoreMesh(core_axis_name="c", subcore_axis_name="s")

    row_buf = pltpu.VMEM((CH, DIM), jnp.float32)

    @functools.partial(
        pl.kernel,
        out_type=jax.ShapeDtypeStruct((NW, L), jnp.float32),
        mesh=mesh,
        compiler_params=pltpu.CompilerParams(needs_layout_passes=False),
        scratch_types=[
            pltpu.VMEM((n_per_w,), jnp.int32),
            pltpu.VMEM((n_per_w,), jnp.int32),
            pltpu.VMEM((n_per_w,), jnp.float32),
            pltpu.VMEM((8,), jnp.int32),
            pltpu.VMEM((8, DIM), jnp.float32),
            [row_buf] * NBUF,
            [row_buf] * NBUF,
            pltpu.VMEM((L, L), jnp.float32),
            [pltpu.SemaphoreType.DMA] * NBUF,
            [pltpu.SemaphoreType.DMA] * NBUF,
            pltpu.SemaphoreType.DMA,
        ],
    )
    def sc_kernel(table_hbm, pconf_hbm, aidx_hbm, pos_hbm, neg_hbm, out_hbm,
                  posidx_v, negidx_v, pconf_v, aidx_v, a_v,
                  posbufs, negbufs, comb_v, psems, nsems, sem_s):
        wid = lax.axis_index("s") * NC + lax.axis_index("c")
        base = wid * n_per_w

        ci = pltpu.async_copy(pos_hbm.at[pl.ds(base, n_per_w)], posidx_v, sem_s)
        cj = pltpu.async_copy(neg_hbm.at[pl.ds(base, n_per_w)], negidx_v, sem_s)
        ck = pltpu.async_copy(pconf_hbm.at[pl.ds(base, n_per_w)], pconf_v, sem_s)
        pltpu.sync_copy(aidx_hbm, aidx_v)
        ca = pltpu.async_copy(table_hbm.at[aidx_v], a_v, sem_s)
        ci.wait()
        cj.wait()
        ck.wait()
        ca.wait()

        def start_gather(c):
            b = c % NBUF
            cp = pltpu.async_copy(
                table_hbm.at[posidx_v.at[pl.ds(c * CH, CH)]], posbufs[b],
                psems[b])
            cn = pltpu.async_copy(
                table_hbm.at[negidx_v.at[pl.ds(c * CH, CH)]], negbufs[b],
                nsems[b])
            return cp, cn

        pending = {0: start_gather(0)}
        a_sl = [a_v[0, pl.ds(L * j, L)] for j in range(NJ)]

        lane = jnp.arange(L, dtype=jnp.int32)
        loss_vec = jnp.zeros((L,), jnp.float32)
        for c in range(n_chunks):
            cp, cn = pending.pop(c)
            cp.wait()
            cn.wait()
            if c + 1 < n_chunks:
                pending[c + 1] = start_gather(c + 1)
            posrows_v = posbufs[c % NBUF]
            negrows_v = negbufs[c % NBUF]

            def group_body(g, lvec, posrows_v=posrows_v, negrows_v=negrows_v,
                           c=c):
                base_r = g * L
                for r in range(L):
                    rr = base_r + r
                    pacc = jnp.abs(a_sl[0] - posrows_v[rr, pl.ds(0, L)])
                    nacc = jnp.abs(a_sl[0] - negrows_v[rr, pl.ds(0, L)])
                    for j in range(1, NJ):
                        pacc = pacc + jnp.abs(
                            a_sl[j] - posrows_v[rr, pl.ds(L * j, L)])
                        nacc = nacc + jnp.abs(
                            a_sl[j] - negrows_v[rr, pl.ds(L * j, L)])
                    pcs = plsc.load_gather(
                        pconf_v, [jnp.full((L,), c * CH + rr, jnp.int32)])
                    comb_v[r] = pcs * pacc - nacc
                colsum = plsc.load_gather(
                    comb_v, [lane, jnp.zeros((L,), jnp.int32)])
                for j in range(1, L):
                    colsum = colsum + plsc.load_gather(
                        comb_v, [lane, jnp.full((L,), j, jnp.int32)])
                return lvec + jnp.maximum(GAMMA + colsum, jnp.float32(0.0))

            loss_vec = lax.fori_loop(0, n_groups, group_body, loss_vec)

        comb_v[0] = loss_vec
        pltpu.sync_copy(comb_v.at[0], out_hbm.at[wid])

    return sc_kernel


def kernel(rel_emb, pconfi, rel_a, rel_pos, rel_neg):
    R = rel_pos.shape[0]
    sc = _build_sc(R)
    aidx = jnp.full((8,), rel_a, jnp.int32)
    partials = sc(rel_emb, pconfi, aidx,
                  rel_pos.astype(jnp.int32), rel_neg.astype(jnp.int32))
    return jnp.sum(partials)
